# baseline (device time: 19336 ns/iter reference)
import jax
import jax.numpy as jnp
from jax import lax
from jax.experimental import pallas as pl
from jax.experimental.pallas import tpu as pltpu


def kernel(ids, E):
    v_per, d = E.shape
    t = ids.shape[0]

    my_y = lax.axis_index("y")
    local = ids - my_y * v_per
    valid = (local >= 0) & (local < v_per)
    safe = jnp.where(valid, local, 0)
    partial = jnp.where(valid[:, None], E[safe, :], jnp.float32(0.0))

    def body(p_ref, out_ref, comm_ref, send_sem, recv_sem):
        mx = lax.axis_index("x")
        my = lax.axis_index("y")
        mz = lax.axis_index("z")
        nbr = (mx, 1 - my, mz)

        barrier = pltpu.get_barrier_semaphore()
        pl.semaphore_signal(
            barrier, inc=1, device_id=nbr,
            device_id_type=pl.DeviceIdType.MESH,
        )
        pl.semaphore_wait(barrier, 1)

        rdma = pltpu.make_async_remote_copy(
            src_ref=p_ref,
            dst_ref=comm_ref,
            send_sem=send_sem,
            recv_sem=recv_sem,
            device_id=nbr,
            device_id_type=pl.DeviceIdType.MESH,
        )
        rdma.start()
        rdma.wait()
        out_ref[:, :] = p_ref[:, :] + comm_ref[:, :]

    return pl.pallas_call(
        body,
        out_shape=jax.ShapeDtypeStruct((t, d), jnp.float32),
        in_specs=[pl.BlockSpec(memory_space=pltpu.VMEM)],
        out_specs=pl.BlockSpec(memory_space=pltpu.VMEM),
        scratch_shapes=[
            pltpu.VMEM((t, d), jnp.float32),
            pltpu.SemaphoreType.DMA,
            pltpu.SemaphoreType.DMA,
        ],
        compiler_params=pltpu.CompilerParams(collective_id=0),
    )(partial)


# device time: 17189 ns/iter; 1.1249x vs baseline; 1.1249x over previous
import jax
import jax.numpy as jnp
from jax import lax
from jax.experimental import pallas as pl
from jax.experimental.pallas import tpu as pltpu


def kernel(ids, E):
    v_per, d = E.shape
    t = ids.shape[0]
    q = t // 4

    my_y = lax.axis_index("y")
    local = ids - my_y * v_per
    valid = (local >= 0) & (local < v_per)
    safe = jnp.where(valid, local, 0)
    partial = jnp.where(valid[:, None], E[safe, :], jnp.float32(0.0))

    def body(p_ref, out_ref, fresh_ref, rx_ref, send_sems, recv_sems):
        mx = lax.axis_index("x")
        my = lax.axis_index("y")
        mz = lax.axis_index("z")
        y_nbr = (mx, 1 - my, mz)
        x_nbr = (1 - mx, my, mz)
        z_nbr = (mx, my, 1 - mz)

        a = 2 * mx + mz
        b = 3 - a
        qa = pl.ds(a * q, q)
        qb = pl.ds(b * q, q)

        barrier = pltpu.get_barrier_semaphore()
        for nbr in (y_nbr, x_nbr, z_nbr):
            pl.semaphore_signal(
                barrier, inc=1, device_id=nbr,
                device_id_type=pl.DeviceIdType.MESH,
            )
        pl.semaphore_wait(barrier, 3)

        send_a = pltpu.make_async_remote_copy(
            src_ref=p_ref.at[qa, :], dst_ref=fresh_ref.at[0],
            send_sem=send_sems.at[0], recv_sem=recv_sems.at[0],
            device_id=y_nbr, device_id_type=pl.DeviceIdType.MESH,
        )
        send_b = pltpu.make_async_remote_copy(
            src_ref=p_ref.at[qb, :], dst_ref=fresh_ref.at[1],
            send_sem=send_sems.at[1], recv_sem=recv_sems.at[1],
            device_id=y_nbr, device_id_type=pl.DeviceIdType.MESH,
        )
        send_a.start()
        send_b.start()

        send_a.wait_recv()
        out_ref[qa, :] = p_ref[qa, :] + fresh_ref[0, :, :]
        fwd_x = pltpu.make_async_remote_copy(
            src_ref=out_ref.at[qa, :], dst_ref=rx_ref.at[0],
            send_sem=send_sems.at[2], recv_sem=recv_sems.at[2],
            device_id=x_nbr, device_id_type=pl.DeviceIdType.MESH,
        )
        fwd_z = pltpu.make_async_remote_copy(
            src_ref=out_ref.at[qa, :], dst_ref=rx_ref.at[1],
            send_sem=send_sems.at[3], recv_sem=recv_sems.at[3],
            device_id=z_nbr, device_id_type=pl.DeviceIdType.MESH,
        )
        fwd_x.start()
        fwd_z.start()

        send_b.wait_recv()
        out_ref[qb, :] = p_ref[qb, :] + fresh_ref[1, :, :]

        fwd_x.wait_recv()
        out_ref[pl.ds((2 * (1 - mx) + mz) * q, q), :] = rx_ref[0, :, :]
        fwd_z.wait_recv()
        out_ref[pl.ds((2 * mx + (1 - mz)) * q, q), :] = rx_ref[1, :, :]

        send_a.wait_send()
        send_b.wait_send()
        fwd_x.wait_send()
        fwd_z.wait_send()

    return pl.pallas_call(
        body,
        out_shape=jax.ShapeDtypeStruct((t, d), jnp.float32),
        in_specs=[pl.BlockSpec(memory_space=pltpu.VMEM)],
        out_specs=pl.BlockSpec(memory_space=pltpu.VMEM),
        scratch_shapes=[
            pltpu.VMEM((2, q, d), jnp.float32),
            pltpu.VMEM((2, q, d), jnp.float32),
            pltpu.SemaphoreType.DMA((4,)),
            pltpu.SemaphoreType.DMA((4,)),
        ],
        compiler_params=pltpu.CompilerParams(collective_id=0),
    )(partial)


# device time: 16583 ns/iter; 1.1660x vs baseline; 1.0365x over previous
import jax
import jax.numpy as jnp
from jax import lax
from jax.experimental import pallas as pl
from jax.experimental.pallas import tpu as pltpu


def kernel(ids, E):
    v_per, d = E.shape
    t = ids.shape[0]
    q = t // 4

    my_y = lax.axis_index("y")
    local = (ids - my_y * v_per).astype(jnp.int32)
    raw = E[jnp.bitwise_and(local, v_per - 1), :]

    def body(p_ref, l_ref, out_ref, fresh_ref, send_sems, recv_sems):
        mx = lax.axis_index("x")
        my = lax.axis_index("y")
        mz = lax.axis_index("z")
        y_nbr = (mx, 1 - my, mz)
        x_nbr = (1 - mx, my, mz)
        z_nbr = (mx, my, 1 - mz)

        a = 2 * mx + mz
        b = 3 - a
        qa = pl.ds(a * q, q)
        qb = pl.ds(b * q, q)

        barrier = pltpu.get_barrier_semaphore()
        for nbr in (y_nbr, x_nbr, z_nbr):
            pl.semaphore_signal(
                barrier, inc=1, device_id=nbr,
                device_id_type=pl.DeviceIdType.MESH,
            )
        pl.semaphore_wait(barrier, 3)

        send_a = pltpu.make_async_remote_copy(
            src_ref=p_ref.at[qa, :], dst_ref=fresh_ref.at[0],
            send_sem=send_sems.at[0], recv_sem=recv_sems.at[0],
            device_id=y_nbr, device_id_type=pl.DeviceIdType.MESH,
        )
        send_b = pltpu.make_async_remote_copy(
            src_ref=p_ref.at[qb, :], dst_ref=fresh_ref.at[1],
            send_sem=send_sems.at[1], recv_sem=recv_sems.at[1],
            device_id=y_nbr, device_id_type=pl.DeviceIdType.MESH,
        )
        send_a.start()
        send_b.start()

        def merge(rows, fresh_slot):
            mine = l_ref[rows, :]
            valid = (mine >= 0) & (mine < v_per)
            out_ref[rows, :] = jnp.where(
                valid, p_ref[rows, :], fresh_ref[fresh_slot, :, :]
            )

        send_a.wait_recv()
        merge(qa, 0)
        fwd_x = pltpu.make_async_remote_copy(
            src_ref=out_ref.at[qa, :], dst_ref=out_ref.at[qa, :],
            send_sem=send_sems.at[2], recv_sem=recv_sems.at[2],
            device_id=x_nbr, device_id_type=pl.DeviceIdType.MESH,
        )
        fwd_z = pltpu.make_async_remote_copy(
            src_ref=out_ref.at[qa, :], dst_ref=out_ref.at[qa, :],
            send_sem=send_sems.at[3], recv_sem=recv_sems.at[3],
            device_id=z_nbr, device_id_type=pl.DeviceIdType.MESH,
        )
        fwd_x.start()
        fwd_z.start()

        send_b.wait_recv()
        merge(qb, 1)

        fwd_x.wait_recv()
        fwd_z.wait_recv()

        send_a.wait_send()
        send_b.wait_send()
        fwd_x.wait_send()
        fwd_z.wait_send()

    return pl.pallas_call(
        body,
        out_shape=jax.ShapeDtypeStruct((t, d), jnp.float32),
        in_specs=[
            pl.BlockSpec(memory_space=pltpu.VMEM),
            pl.BlockSpec(memory_space=pltpu.VMEM),
        ],
        out_specs=pl.BlockSpec(memory_space=pltpu.VMEM),
        scratch_shapes=[
            pltpu.VMEM((2, q, d), jnp.float32),
            pltpu.SemaphoreType.DMA((4,)),
            pltpu.SemaphoreType.DMA((4,)),
        ],
        compiler_params=pltpu.CompilerParams(collective_id=0),
    )(raw, local[:, None])


# device time: 16501 ns/iter; 1.1718x vs baseline; 1.0050x over previous
import jax
import jax.numpy as jnp
from jax import lax
from jax.experimental import pallas as pl
from jax.experimental.pallas import tpu as pltpu


def kernel(ids, E):
    v_per, d = E.shape
    t = ids.shape[0]
    q = t // 4
    h = q // 2

    my_x = lax.axis_index("x")
    my_y = lax.axis_index("y")
    my_z = lax.axis_index("z")
    a = 2 * my_x + my_z
    b = 3 - a

    local = (ids - my_y * v_per).astype(jnp.int32)
    rows = jnp.concatenate(
        [a * q + jnp.arange(q, dtype=jnp.int32),
         b * q + jnp.arange(q, dtype=jnp.int32)]
    )
    lsel = local[rows]
    raw = E[jnp.bitwise_and(lsel, v_per - 1), :]

    def body(p_ref, l_ref, out_ref, fresh_ref, y_ssem, y_rsem, f_ssem, f_rsem):
        mx = lax.axis_index("x")
        my = lax.axis_index("y")
        mz = lax.axis_index("z")
        y_nbr = (mx, 1 - my, mz)
        x_nbr = (1 - mx, my, mz)
        z_nbr = (mx, my, 1 - mz)
        aa = 2 * mx + mz

        barrier = pltpu.get_barrier_semaphore()
        for nbr in (y_nbr, x_nbr, z_nbr):
            pl.semaphore_signal(
                barrier, inc=1, device_id=nbr,
                device_id_type=pl.DeviceIdType.MESH,
            )
        pl.semaphore_wait(barrier, 3)

        y_sends = []
        for i, (off, ln) in enumerate(((0, h), (h, h), (q, q))):
            snd = pltpu.make_async_remote_copy(
                src_ref=p_ref.at[pl.ds(off, ln), :],
                dst_ref=fresh_ref.at[pl.ds(off, ln), :],
                send_sem=y_ssem.at[i], recv_sem=y_rsem.at[i],
                device_id=y_nbr, device_id_type=pl.DeviceIdType.MESH,
            )
            snd.start()
            y_sends.append(snd)

        def merge(off, ln, out_off):
            mine = l_ref[pl.ds(off, ln), :]
            valid = (mine >= 0) & (mine < v_per)
            out_ref[pl.ds(out_off, ln), :] = jnp.where(
                valid,
                p_ref[pl.ds(off, ln), :],
                fresh_ref[pl.ds(off, ln), :],
            )

        fwds = []
        for c, snd in enumerate(y_sends[:2]):
            snd.wait_recv()
            off = c * h
            merge(off, h, aa * q + off)
            for j, nbr in enumerate((x_nbr, z_nbr)):
                k = 2 * c + j
                fwd = pltpu.make_async_remote_copy(
                    src_ref=out_ref.at[pl.ds(aa * q + off, h), :],
                    dst_ref=out_ref.at[pl.ds(aa * q + off, h), :],
                    send_sem=f_ssem.at[k], recv_sem=f_rsem.at[k],
                    device_id=nbr, device_id_type=pl.DeviceIdType.MESH,
                )
                fwd.start()
                fwds.append(fwd)

        y_sends[2].wait_recv()
        merge(q, q, (3 - aa) * q)

        for fwd in fwds:
            fwd.wait_recv()
        for snd in y_sends:
            snd.wait_send()
        for fwd in fwds:
            fwd.wait_send()

    return pl.pallas_call(
        body,
        out_shape=jax.ShapeDtypeStruct((t, d), jnp.float32),
        in_specs=[
            pl.BlockSpec(memory_space=pltpu.VMEM),
            pl.BlockSpec(memory_space=pltpu.VMEM),
        ],
        out_specs=pl.BlockSpec(memory_space=pltpu.VMEM),
        scratch_shapes=[
            pltpu.VMEM((2 * q, d), jnp.float32),
            pltpu.SemaphoreType.DMA((3,)),
            pltpu.SemaphoreType.DMA((3,)),
            pltpu.SemaphoreType.DMA((4,)),
            pltpu.SemaphoreType.DMA((4,)),
        ],
        compiler_params=pltpu.CompilerParams(collective_id=0),
    )(raw, lsel[:, None])
